# baseline (speedup 1.0000x reference)
import functools

import jax
import jax.numpy as jnp
from jax import lax
from jax.experimental import pallas as pl
from jax.experimental.pallas import tpu as pltpu
from jax.experimental.pallas import tpu_sc as plsc

_B = 1024
_M = 256
_D = 128

_info = plsc.get_sparse_core_info()
_NC, _NS, _L = _info.num_cores, _info.num_subcores, _info.num_lanes
_NW = _NC * _NS
_CH = _B // _NW

_mesh = plsc.VectorSubcoreMesh(core_axis_name="c", subcore_axis_name="s")


@functools.partial(
    pl.kernel,
    mesh=_mesh,
    out_type=[
        jax.ShapeDtypeStruct((_B * _M, _D), jnp.float32),
        jax.ShapeDtypeStruct((_B,), jnp.int32),
    ],
    scratch_types=[
        pltpu.VMEM((_CH,), jnp.int32),
        pltpu.VMEM((_CH,), jnp.int32),
        pltpu.VMEM((_CH,), jnp.int32),
        pltpu.VMEM((_CH, _D), jnp.float32),
        pltpu.SemaphoreType.DMA,
        pltpu.SemaphoreType.DMA,
    ],
)
def _sc_body(z_hbm, mem_hbm, state_hbm, out_hbm, ctr_hbm,
             state_v, idx_v, ctr_v, z_v, sem_copy, sem_io):
    wid = lax.axis_index("s") * _NC + lax.axis_index("c")
    base_b = wid * _CH
    row0 = base_b * _M
    nrows = _CH * _M

    copy = pltpu.make_async_copy(
        mem_hbm.at[pl.ds(row0, nrows)], out_hbm.at[pl.ds(row0, nrows)],
        sem_copy)
    copy.start()

    pltpu.sync_copy(state_hbm.at[pl.ds(base_b, _CH)], state_v)
    pltpu.sync_copy(z_hbm.at[pl.ds(base_b, _CH)], z_v)

    for k in range(_CH // _L):
        s = state_v[pl.ds(k * _L, _L)]
        ctr_v[pl.ds(k * _L, _L)] = s + 1
        fb = (base_b + k * _L) * _M
        idx_v[pl.ds(k * _L, _L)] = (
            fb + lax.iota(jnp.int32, _L) * _M + lax.rem(s, _M))

    pltpu.sync_copy(ctr_v, ctr_hbm.at[pl.ds(base_b, _CH)])

    copy.wait()
    pltpu.async_copy(z_v, out_hbm.at[idx_v], sem_io).wait()


def kernel(z, mem_state, state):
    b, m, d = mem_state.shape
    mem2d = mem_state.reshape(b * m, d)
    out2d, ctr = _sc_body(z, mem2d, state)
    return out2d.reshape(b, m, d), ctr


# speedup vs baseline: 35.6296x; 35.6296x over previous
import functools

import jax
import jax.numpy as jnp
from jax import lax
from jax.experimental import pallas as pl
from jax.experimental.pallas import tpu as pltpu
from jax.experimental.pallas import tpu_sc as plsc

_B = 1024
_M = 256
_D = 128

_info = plsc.get_sparse_core_info()
_NC, _NS, _L = _info.num_cores, _info.num_subcores, _info.num_lanes
_NW = _NC * _NS
_CH = _B // _NW

_CROWS = 128
_NBUF = 4
_SLAB = _CH * _M
_NCHUNK = _SLAB // _CROWS

_mesh = plsc.VectorSubcoreMesh(core_axis_name="c", subcore_axis_name="s")


@functools.partial(
    pl.kernel,
    mesh=_mesh,
    out_type=[
        jax.ShapeDtypeStruct((_B * _M, _D), jnp.float32),
        jax.ShapeDtypeStruct((_B,), jnp.int32),
    ],
    scratch_types=[
        pltpu.VMEM((_NBUF, _CROWS, _D), jnp.float32),
        pltpu.VMEM((_CH + _L,), jnp.int32),
        pltpu.VMEM((_CH,), jnp.int32),
        pltpu.VMEM((_CH * _D,), jnp.float32),
        pltpu.SemaphoreType.DMA((_NBUF,)),
        pltpu.SemaphoreType.DMA((_NBUF,)),
    ],
)
def _sc_body(z_hbm, mem_hbm, state_hbm, out_hbm, ctr_hbm,
             bufs, state_v, ctr_v, z_v, sem_in, sem_out):
    wid = lax.axis_index("s") * _NC + lax.axis_index("c")
    base_b = wid * _CH
    row0 = base_b * _M

    def start_in(j, s):
        pltpu.make_async_copy(
            mem_hbm.at[pl.ds(row0 + j * _CROWS, _CROWS)],
            bufs.at[s], sem_in.at[s]).start()

    def wait_in(j, s):
        pltpu.make_async_copy(
            mem_hbm.at[pl.ds(row0 + j * _CROWS, _CROWS)],
            bufs.at[s], sem_in.at[s]).wait()

    def start_out(j, s):
        pltpu.make_async_copy(
            bufs.at[s], out_hbm.at[pl.ds(row0 + j * _CROWS, _CROWS)],
            sem_out.at[s]).start()

    def wait_out(j, s):
        pltpu.make_async_copy(
            bufs.at[s], out_hbm.at[pl.ds(row0 + j * _CROWS, _CROWS)],
            sem_out.at[s]).wait()

    pltpu.sync_copy(state_hbm.at[pl.ds(base_b, _CH)], state_v.at[pl.ds(0, _CH)])
    pltpu.sync_copy(z_hbm.at[pl.ds(base_b * _D, _CH * _D)], z_v)

    for s in range(_NBUF):
        start_in(s, s)

    _HALF = _M // _CROWS

    def patch(j, s):
        b = j // _HALF
        r = lax.rem(state_v[pl.ds(b, _L)][0], _M)

        @pl.when(r // _CROWS == lax.rem(j, _HALF))
        def _():
            r_loc = lax.rem(r, _CROWS)
            for c in range(_D // _L):
                bufs[s, r_loc, pl.ds(c * _L, _L)] = (
                    z_v[pl.ds(b * _D + c * _L, _L)])

    @pl.loop(0, (_NCHUNK - _NBUF) // _NBUF)
    def _ring(i):
        j0 = i * _NBUF
        for s in range(_NBUF):
            j = j0 + s
            wait_in(j, s)
            patch(j, s)
            start_out(j, s)
            wait_out(j, s)
            start_in(j + _NBUF, s)

    for s in range(_NBUF):
        j = _NCHUNK - _NBUF + s
        wait_in(j, s)
        patch(j, s)
        start_out(j, s)

    for k in range(_CH // _L):
        sv = state_v[pl.ds(k * _L, _L)]
        ctr_v[pl.ds(k * _L, _L)] = sv + 1

    pltpu.sync_copy(ctr_v, ctr_hbm.at[pl.ds(base_b, _CH)])

    for s in range(_NBUF):
        wait_out(_NCHUNK - _NBUF + s, s)


def kernel(z, mem_state, state):
    b, m, d = mem_state.shape
    mem2d = mem_state.reshape(b * m, d)
    out2d, ctr = _sc_body(z.reshape(b * d), mem2d, state)
    return out2d.reshape(b, m, d), ctr
